# baseline (speedup 1.0000x reference)
import functools

import jax
import jax.numpy as jnp
from jax import lax
from jax.experimental import pallas as pl
from jax.experimental.pallas import tpu as pltpu
from jax.experimental.pallas import tpu_sc as plsc

B, L, T, D = 1024, 50, 20, 128
N = B * L
D3 = 3 * D
NCSEG = 8
HALF = NCSEG // 2
ROWS_H = HALF * T


def _pool_half(rows_v, rrows_v, out_v, s_off):
    @pl.loop(0, HALF)
    def _seg(s):
        base = s * T
        so = s + s_off
        accs = []
        for j in range(D // 16):
            sl = pl.ds(j * 16, 16)
            parent = rows_v[base, sl]
            out_v[so, pl.ds(j * 16, 16)] = parent
            accs.append(parent)
        for t in range(1, T):
            for j in range(D // 16):
                accs[j] = accs[j] + rows_v[base + t, pl.ds(j * 16, 16)]
        for j in range(D // 16):
            out_v[so, pl.ds(D + j * 16, 16)] = accs[j]
            out_v[so, pl.ds(2 * D + j * 16, 16)] = rrows_v[so, pl.ds(j * 16, 16)]


def _sc_body(nseg_w, base_seg, tids_ref, rids_ref, tok_ref, rul_ref, out_ref,
             ids_v, rids_v, rows0a, rows0b, rows1a, rows1b, rr0, rr1,
             outv0, outv1, sem_ids, sem0, sem1, semo0, semo1):
    nc = 2
    wid = lax.axis_index("s") * nc + lax.axis_index("c")
    seg0 = wid * nseg_w
    gseg0 = base_seg + seg0
    nchunks = nseg_w // NCSEG

    cp_t = pltpu.async_copy(tids_ref.at[pl.ds(gseg0 * T, nseg_w * T)], ids_v,
                            sem_ids)
    cp_r = pltpu.async_copy(rids_ref.at[pl.ds(gseg0, nseg_w)], rids_v, sem_ids)
    cp_t.wait()
    cp_r.wait()

    def issue(c, ra, rb, rr, sem):
        off = c * (NCSEG * T)
        pltpu.async_copy(tok_ref.at[ids_v.at[pl.ds(off, ROWS_H)]], ra, sem)
        pltpu.async_copy(tok_ref.at[ids_v.at[pl.ds(off + ROWS_H, ROWS_H)]],
                         rb, sem)
        pltpu.async_copy(rul_ref.at[rids_v.at[pl.ds(c * NCSEG, NCSEG)]], rr,
                         sem)

    def drain(ra, rb, rr, sem):
        pltpu.make_async_copy(tok_ref.at[pl.ds(0, ROWS_H)], ra, sem).wait()
        pltpu.make_async_copy(tok_ref.at[pl.ds(0, ROWS_H)], rb, sem).wait()
        pltpu.make_async_copy(rul_ref.at[pl.ds(0, NCSEG)], rr, sem).wait()

    def wait_out(outv, semo):
        pltpu.make_async_copy(outv, out_ref.at[pl.ds(0, NCSEG)], semo).wait()

    def pool_store(c, ra, rb, rr, outv, semo):
        _pool_half(ra, rr, outv, 0)
        _pool_half(rb, rr, outv, HALF)
        pltpu.async_copy(outv, out_ref.at[pl.ds(seg0 + c * NCSEG, NCSEG)],
                         semo)

    issue(0, rows0a, rows0b, rr0, sem0)

    @pl.loop(0, nchunks, step=2)
    def _chunk(c):
        issue(c + 1, rows1a, rows1b, rr1, sem1)
        drain(rows0a, rows0b, rr0, sem0)

        @pl.when(c >= 2)
        def _():
            wait_out(outv0, semo0)

        pool_store(c, rows0a, rows0b, rr0, outv0, semo0)

        @pl.when(c + 2 < nchunks)
        def _():
            issue(c + 2, rows0a, rows0b, rr0, sem0)

        drain(rows1a, rows1b, rr1, sem1)

        @pl.when(c >= 2)
        def _():
            wait_out(outv1, semo1)

        pool_store(c + 1, rows1a, rows1b, rr1, outv1, semo1)

    wait_out(outv0, semo0)
    wait_out(outv1, semo1)


def _gather_pool(tids, rids, token_embeds, rule_embeds, nseg, base_seg):
    info = plsc.get_sparse_core_info()
    nw = info.num_cores * info.num_subcores
    nseg_w = nseg // nw
    mesh = plsc.VectorSubcoreMesh(core_axis_name="c", subcore_axis_name="s")
    kfn = pl.kernel(
        functools.partial(_sc_body, nseg_w, base_seg),
        out_type=jax.ShapeDtypeStruct((nseg, D3), jnp.float32),
        mesh=mesh,
        scratch_types=[
            pltpu.VMEM((nseg_w * T,), jnp.int32),
            pltpu.VMEM((nseg_w,), jnp.int32),
            pltpu.VMEM((ROWS_H, D), jnp.float32),
            pltpu.VMEM((ROWS_H, D), jnp.float32),
            pltpu.VMEM((ROWS_H, D), jnp.float32),
            pltpu.VMEM((ROWS_H, D), jnp.float32),
            pltpu.VMEM((NCSEG, D), jnp.float32),
            pltpu.VMEM((NCSEG, D), jnp.float32),
            pltpu.VMEM((NCSEG, D3), jnp.float32),
            pltpu.VMEM((NCSEG, D3), jnp.float32),
            pltpu.SemaphoreType.DMA,
            pltpu.SemaphoreType.DMA,
            pltpu.SemaphoreType.DMA,
            pltpu.SemaphoreType.DMA,
            pltpu.SemaphoreType.DMA,
        ],
        compiler_params=pltpu.CompilerParams(use_tc_tiling_on_sc=True),
    )
    return kfn(tids, rids, token_embeds, rule_embeds)


MM_BB = 16
NSLICE = 4


def _mm_compute(x_ref, wt_ref, b_ref, o_ref):
    wt = wt_ref[...]
    bias = b_ref[...]
    for j in range(MM_BB):
        x = jnp.maximum(x_ref[pl.ds(j * L, L), :], 0.0)
        o_ref[j] = (
            jnp.dot(x, wt, preferred_element_type=jnp.float32) + bias
        )


def _mm_body(x_ref, wt_ref, b_ref, o_ref):
    _mm_compute(x_ref, wt_ref, b_ref, o_ref)


def _mm_body_alias(x_ref, wt_ref, b_ref, prev_ref, o_ref):
    del prev_ref
    _mm_compute(x_ref, wt_ref, b_ref, o_ref)


def _project_slice(x, wt, b2, bs):
    nblk = bs // MM_BB
    return pl.pallas_call(
        _mm_body,
        grid=(nblk,),
        in_specs=[
            pl.BlockSpec((MM_BB * L, D3), lambda i: (i, 0)),
            pl.BlockSpec((D3, D), lambda i: (0, 0)),
            pl.BlockSpec((1, D), lambda i: (0, 0)),
        ],
        out_specs=pl.BlockSpec((MM_BB, L, D), lambda i: (i, 0, 0)),
        out_shape=jax.ShapeDtypeStruct((bs, L, D), jnp.float32),
    )(x, wt, b2)


def kernel(rule_token_ids, rule_ids, token_embeds, rule_embeds, W, b):
    bs = B // NSLICE
    wt = W.T
    b2 = b.reshape(1, D)
    outs = []
    for s in range(NSLICE):
        tids = rule_token_ids[s * bs:(s + 1) * bs].reshape(-1)
        rids = rule_ids[s * bs:(s + 1) * bs].reshape(-1)
        concat = _gather_pool(tids.astype(jnp.int32), rids.astype(jnp.int32),
                              token_embeds, rule_embeds, bs * L, 0)
        outs.append(_project_slice(concat, wt, b2, bs))
    return jnp.concatenate(outs, axis=0)


# speedup vs baseline: 1.0632x; 1.0632x over previous
import functools

import jax
import jax.numpy as jnp
from jax import lax
from jax.experimental import pallas as pl
from jax.experimental.pallas import tpu as pltpu
from jax.experimental.pallas import tpu_sc as plsc

B, L, T, D = 1024, 50, 20, 128
N = B * L
D3 = 3 * D
NCSEG = 8
HALF = NCSEG // 2
ROWS_H = HALF * T


def _pool_half(rows_v, rrows_v, out_v, s_off):
    @pl.loop(0, HALF)
    def _seg(s):
        base = s * T
        so = s + s_off
        accs = []
        for j in range(D // 16):
            sl = pl.ds(j * 16, 16)
            parent = rows_v[base, sl]
            out_v[so, pl.ds(j * 16, 16)] = parent
            accs.append(parent)
        for t in range(1, T):
            for j in range(D // 16):
                accs[j] = accs[j] + rows_v[base + t, pl.ds(j * 16, 16)]
        for j in range(D // 16):
            out_v[so, pl.ds(D + j * 16, 16)] = accs[j]
            out_v[so, pl.ds(2 * D + j * 16, 16)] = rrows_v[so, pl.ds(j * 16, 16)]


def _sc_body(nseg_w, base_seg, tids_ref, rids_ref, tok_ref, rul_ref, out_ref,
             ids_v, rids_v, rows0a, rows0b, rows1a, rows1b, rr0, rr1,
             outv0, outv1, sem_ids, sem0, sem1, semo0, semo1):
    nc = 2
    wid = lax.axis_index("s") * nc + lax.axis_index("c")
    seg0 = wid * nseg_w
    gseg0 = base_seg + seg0
    nchunks = nseg_w // NCSEG

    cp_t = pltpu.async_copy(tids_ref.at[pl.ds(gseg0 * T, nseg_w * T)], ids_v,
                            sem_ids)
    cp_r = pltpu.async_copy(rids_ref.at[pl.ds(gseg0, nseg_w)], rids_v, sem_ids)
    cp_t.wait()
    cp_r.wait()

    def issue(c, ra, rb, rr, sem):
        off = c * (NCSEG * T)
        pltpu.async_copy(tok_ref.at[ids_v.at[pl.ds(off, ROWS_H)]], ra, sem)
        pltpu.async_copy(tok_ref.at[ids_v.at[pl.ds(off + ROWS_H, ROWS_H)]],
                         rb, sem)
        pltpu.async_copy(rul_ref.at[rids_v.at[pl.ds(c * NCSEG, NCSEG)]], rr,
                         sem)

    def drain(ra, rb, rr, sem):
        pltpu.make_async_copy(tok_ref.at[pl.ds(0, ROWS_H)], ra, sem).wait()
        pltpu.make_async_copy(tok_ref.at[pl.ds(0, ROWS_H)], rb, sem).wait()
        pltpu.make_async_copy(rul_ref.at[pl.ds(0, NCSEG)], rr, sem).wait()

    def wait_out(outv, semo):
        pltpu.make_async_copy(outv, out_ref.at[pl.ds(0, NCSEG)], semo).wait()

    def pool_store(c, ra, rb, rr, outv, semo):
        _pool_half(ra, rr, outv, 0)
        _pool_half(rb, rr, outv, HALF)
        pltpu.async_copy(outv, out_ref.at[pl.ds(seg0 + c * NCSEG, NCSEG)],
                         semo)

    issue(0, rows0a, rows0b, rr0, sem0)

    @pl.loop(0, nchunks, step=2)
    def _chunk(c):
        issue(c + 1, rows1a, rows1b, rr1, sem1)
        drain(rows0a, rows0b, rr0, sem0)

        @pl.when(c >= 2)
        def _():
            wait_out(outv0, semo0)

        pool_store(c, rows0a, rows0b, rr0, outv0, semo0)

        @pl.when(c + 2 < nchunks)
        def _():
            issue(c + 2, rows0a, rows0b, rr0, sem0)

        drain(rows1a, rows1b, rr1, sem1)

        @pl.when(c >= 2)
        def _():
            wait_out(outv1, semo1)

        pool_store(c + 1, rows1a, rows1b, rr1, outv1, semo1)

    wait_out(outv0, semo0)
    wait_out(outv1, semo1)


def _gather_pool(tids, rids, token_embeds, rule_embeds, nseg, base_seg):
    info = plsc.get_sparse_core_info()
    nw = info.num_cores * info.num_subcores
    nseg_w = nseg // nw
    mesh = plsc.VectorSubcoreMesh(core_axis_name="c", subcore_axis_name="s")
    kfn = pl.kernel(
        functools.partial(_sc_body, nseg_w, base_seg),
        out_type=jax.ShapeDtypeStruct((nseg, D3), jnp.float32),
        mesh=mesh,
        scratch_types=[
            pltpu.VMEM((nseg_w * T,), jnp.int32),
            pltpu.VMEM((nseg_w,), jnp.int32),
            pltpu.VMEM((ROWS_H, D), jnp.float32),
            pltpu.VMEM((ROWS_H, D), jnp.float32),
            pltpu.VMEM((ROWS_H, D), jnp.float32),
            pltpu.VMEM((ROWS_H, D), jnp.float32),
            pltpu.VMEM((NCSEG, D), jnp.float32),
            pltpu.VMEM((NCSEG, D), jnp.float32),
            pltpu.VMEM((NCSEG, D3), jnp.float32),
            pltpu.VMEM((NCSEG, D3), jnp.float32),
            pltpu.SemaphoreType.DMA,
            pltpu.SemaphoreType.DMA,
            pltpu.SemaphoreType.DMA,
            pltpu.SemaphoreType.DMA,
            pltpu.SemaphoreType.DMA,
        ],
        compiler_params=pltpu.CompilerParams(use_tc_tiling_on_sc=True),
    )
    return kfn(tids, rids, token_embeds, rule_embeds)


MM_BB = 16
NSLICE = 4


def _mm_compute(x_ref, wt_ref, b_ref, o_ref):
    wt = wt_ref[...]
    bias = b_ref[...]
    for j in range(MM_BB):
        x = jnp.maximum(x_ref[pl.ds(j * L, L), :], 0.0)
        o_ref[j] = (
            jnp.dot(x, wt, preferred_element_type=jnp.float32) + bias
        )


def _mm_body(x_ref, wt_ref, b_ref, o_ref):
    _mm_compute(x_ref, wt_ref, b_ref, o_ref)


def _mm_body_alias(x_ref, wt_ref, b_ref, prev_ref, o_ref):
    del prev_ref
    _mm_compute(x_ref, wt_ref, b_ref, o_ref)


def _project_slice(x, wt, b2, prev, s, bs):
    nblk = bs // MM_BB
    in_specs = [
        pl.BlockSpec((MM_BB * L, D3), lambda i: (i, 0)),
        pl.BlockSpec((D3, D), lambda i: (0, 0)),
        pl.BlockSpec((1, D), lambda i: (0, 0)),
    ]
    args = (x, wt, b2)
    kwargs = {}
    body = _mm_body
    if prev is not None:
        in_specs.append(pl.BlockSpec(memory_space=pl.ANY))
        args = args + (prev,)
        kwargs["input_output_aliases"] = {3: 0}
        body = _mm_body_alias
    return pl.pallas_call(
        body,
        grid=(nblk,),
        in_specs=in_specs,
        out_specs=pl.BlockSpec(
            (MM_BB, L, D), lambda i, s=s, nblk=nblk: (i + s * nblk, 0, 0)
        ),
        out_shape=jax.ShapeDtypeStruct((B, L, D), jnp.float32),
        **kwargs,
    )(*args)


def kernel(rule_token_ids, rule_ids, token_embeds, rule_embeds, W, b):
    bs = B // NSLICE
    wt = W.T
    b2 = b.reshape(1, D)
    out = None
    for s in range(NSLICE):
        tids = rule_token_ids[s * bs:(s + 1) * bs].reshape(-1)
        rids = rule_ids[s * bs:(s + 1) * bs].reshape(-1)
        concat = _gather_pool(tids.astype(jnp.int32), rids.astype(jnp.int32),
                              token_embeds, rule_embeds, bs * L, 0)
        out = _project_slice(concat, wt, b2, out, s, bs)
    return out


# speedup vs baseline: 1.0738x; 1.0100x over previous
import functools

import jax
import jax.numpy as jnp
from jax import lax
from jax.experimental import pallas as pl
from jax.experimental.pallas import tpu as pltpu
from jax.experimental.pallas import tpu_sc as plsc

B, L, T, D = 1024, 50, 20, 128
N = B * L
D3 = 3 * D
NCSEG = 8
HALF = NCSEG // 2
ROWS_H = HALF * T


def _pool_half(rows_v, rrows_v, out_v, s_off):
    @pl.loop(0, HALF)
    def _seg(s):
        base = s * T
        so = s + s_off
        accs = []
        for j in range(D // 16):
            sl = pl.ds(j * 16, 16)
            parent = rows_v[base, sl]
            out_v[so, pl.ds(j * 16, 16)] = parent
            accs.append(parent)
        for t in range(1, T):
            for j in range(D // 16):
                accs[j] = accs[j] + rows_v[base + t, pl.ds(j * 16, 16)]
        for j in range(D // 16):
            out_v[so, pl.ds(D + j * 16, 16)] = accs[j]
            out_v[so, pl.ds(2 * D + j * 16, 16)] = rrows_v[so, pl.ds(j * 16, 16)]


def _sc_body(nseg_w, base_seg, tids_ref, rids_ref, tok_ref, rul_ref, out_ref,
             ids_v, rids_v, rows0a, rows0b, rows1a, rows1b, rr0, rr1,
             outv0, outv1, sem_ids, sem0, sem1, semo0, semo1):
    nc = 2
    wid = lax.axis_index("s") * nc + lax.axis_index("c")
    seg0 = wid * nseg_w
    gseg0 = base_seg + seg0
    nchunks = nseg_w // NCSEG

    cp_t = pltpu.async_copy(tids_ref.at[pl.ds(gseg0 * T, nseg_w * T)], ids_v,
                            sem_ids)
    cp_r = pltpu.async_copy(rids_ref.at[pl.ds(gseg0, nseg_w)], rids_v, sem_ids)
    cp_t.wait()
    cp_r.wait()

    def issue(c, ra, rb, rr, sem):
        off = c * (NCSEG * T)
        pltpu.async_copy(tok_ref.at[ids_v.at[pl.ds(off, ROWS_H)]], ra, sem)
        pltpu.async_copy(tok_ref.at[ids_v.at[pl.ds(off + ROWS_H, ROWS_H)]],
                         rb, sem)
        pltpu.async_copy(rul_ref.at[rids_v.at[pl.ds(c * NCSEG, NCSEG)]], rr,
                         sem)

    def drain(ra, rb, rr, sem):
        pltpu.make_async_copy(tok_ref.at[pl.ds(0, ROWS_H)], ra, sem).wait()
        pltpu.make_async_copy(tok_ref.at[pl.ds(0, ROWS_H)], rb, sem).wait()
        pltpu.make_async_copy(rul_ref.at[pl.ds(0, NCSEG)], rr, sem).wait()

    def wait_out(outv, semo):
        pltpu.make_async_copy(outv, out_ref.at[pl.ds(0, NCSEG)], semo).wait()

    def pool_store(c, ra, rb, rr, outv, semo):
        _pool_half(ra, rr, outv, 0)
        _pool_half(rb, rr, outv, HALF)
        pltpu.async_copy(outv, out_ref.at[pl.ds(seg0 + c * NCSEG, NCSEG)],
                         semo)

    issue(0, rows0a, rows0b, rr0, sem0)

    @pl.loop(0, nchunks, step=2)
    def _chunk(c):
        issue(c + 1, rows1a, rows1b, rr1, sem1)
        drain(rows0a, rows0b, rr0, sem0)

        @pl.when(c >= 2)
        def _():
            wait_out(outv0, semo0)

        pool_store(c, rows0a, rows0b, rr0, outv0, semo0)

        @pl.when(c + 2 < nchunks)
        def _():
            issue(c + 2, rows0a, rows0b, rr0, sem0)

        drain(rows1a, rows1b, rr1, sem1)

        @pl.when(c >= 2)
        def _():
            wait_out(outv1, semo1)

        pool_store(c + 1, rows1a, rows1b, rr1, outv1, semo1)

    wait_out(outv0, semo0)
    wait_out(outv1, semo1)


def _gather_pool(tids, rids, token_embeds, rule_embeds, nseg, base_seg):
    info = plsc.get_sparse_core_info()
    nw = info.num_cores * info.num_subcores
    nseg_w = nseg // nw
    mesh = plsc.VectorSubcoreMesh(core_axis_name="c", subcore_axis_name="s")
    kfn = pl.kernel(
        functools.partial(_sc_body, nseg_w, base_seg),
        out_type=jax.ShapeDtypeStruct((nseg, D3), jnp.float32),
        mesh=mesh,
        scratch_types=[
            pltpu.VMEM((nseg_w * T,), jnp.int32),
            pltpu.VMEM((nseg_w,), jnp.int32),
            pltpu.VMEM((ROWS_H, D), jnp.float32),
            pltpu.VMEM((ROWS_H, D), jnp.float32),
            pltpu.VMEM((ROWS_H, D), jnp.float32),
            pltpu.VMEM((ROWS_H, D), jnp.float32),
            pltpu.VMEM((NCSEG, D), jnp.float32),
            pltpu.VMEM((NCSEG, D), jnp.float32),
            pltpu.VMEM((NCSEG, D3), jnp.float32),
            pltpu.VMEM((NCSEG, D3), jnp.float32),
            pltpu.SemaphoreType.DMA,
            pltpu.SemaphoreType.DMA,
            pltpu.SemaphoreType.DMA,
            pltpu.SemaphoreType.DMA,
            pltpu.SemaphoreType.DMA,
        ],
        compiler_params=pltpu.CompilerParams(use_tc_tiling_on_sc=True),
    )
    return kfn(tids, rids, token_embeds, rule_embeds)


MM_BB = 16
NSLICE = 2


def _mm_compute(x_ref, wt_ref, b_ref, o_ref):
    wt = wt_ref[...]
    bias = b_ref[...]
    for j in range(MM_BB):
        x = jnp.maximum(x_ref[pl.ds(j * L, L), :], 0.0)
        o_ref[j] = (
            jnp.dot(x, wt, preferred_element_type=jnp.float32) + bias
        )


def _mm_body(x_ref, wt_ref, b_ref, o_ref):
    _mm_compute(x_ref, wt_ref, b_ref, o_ref)


def _mm_body_alias(x_ref, wt_ref, b_ref, prev_ref, o_ref):
    del prev_ref
    _mm_compute(x_ref, wt_ref, b_ref, o_ref)


def _project_slice(x, wt, b2, prev, s, bs):
    nblk = bs // MM_BB
    in_specs = [
        pl.BlockSpec((MM_BB * L, D3), lambda i: (i, 0)),
        pl.BlockSpec((D3, D), lambda i: (0, 0)),
        pl.BlockSpec((1, D), lambda i: (0, 0)),
    ]
    args = (x, wt, b2)
    kwargs = {}
    body = _mm_body
    if prev is not None:
        in_specs.append(pl.BlockSpec(memory_space=pl.ANY))
        args = args + (prev,)
        kwargs["input_output_aliases"] = {3: 0}
        body = _mm_body_alias
    return pl.pallas_call(
        body,
        grid=(nblk,),
        in_specs=in_specs,
        out_specs=pl.BlockSpec(
            (MM_BB, L, D), lambda i, s=s, nblk=nblk: (i + s * nblk, 0, 0)
        ),
        out_shape=jax.ShapeDtypeStruct((B, L, D), jnp.float32),
        **kwargs,
    )(*args)


def kernel(rule_token_ids, rule_ids, token_embeds, rule_embeds, W, b):
    bs = B // NSLICE
    wt = W.T
    b2 = b.reshape(1, D)
    out = None
    for s in range(NSLICE):
        tids = rule_token_ids[s * bs:(s + 1) * bs].reshape(-1)
        rids = rule_ids[s * bs:(s + 1) * bs].reshape(-1)
        concat = _gather_pool(tids.astype(jnp.int32), rids.astype(jnp.int32),
                              token_embeds, rule_embeds, bs * L, 0)
        out = _project_slice(concat, wt, b2, out, s, bs)
    return out


# speedup vs baseline: 1.0876x; 1.0128x over previous
import functools

import jax
import jax.numpy as jnp
from jax import lax
from jax.experimental import pallas as pl
from jax.experimental.pallas import tpu as pltpu
from jax.experimental.pallas import tpu_sc as plsc

B, L, T, D = 1024, 50, 20, 128
N = B * L
D3 = 3 * D
NCSEG = 8
HALF = NCSEG // 2
ROWS_H = HALF * T


def _pool_half(rows_v, rrows_v, out_v, s_off):
    @pl.loop(0, HALF)
    def _seg(s):
        base = s * T
        so = s + s_off
        accs = []
        for j in range(D // 16):
            sl = pl.ds(j * 16, 16)
            parent = rows_v[base, sl]
            out_v[so, pl.ds(j * 16, 16)] = parent
            accs.append(parent)
        for t in range(1, T):
            for j in range(D // 16):
                accs[j] = accs[j] + rows_v[base + t, pl.ds(j * 16, 16)]
        for j in range(D // 16):
            out_v[so, pl.ds(D + j * 16, 16)] = accs[j]
            out_v[so, pl.ds(2 * D + j * 16, 16)] = rrows_v[so, pl.ds(j * 16, 16)]


def _sc_body(nseg_w, base_seg, tids_ref, rids_ref, tok_ref, rul_ref, out_ref,
             ids_v, rids_v, rows0a, rows0b, rows1a, rows1b, rr0, rr1,
             outv0, outv1, sem_ids, sem0, sem1, semo0, semo1):
    nc = 2
    wid = lax.axis_index("s") * nc + lax.axis_index("c")
    seg0 = wid * nseg_w
    gseg0 = base_seg + seg0
    nchunks = nseg_w // NCSEG

    cp_t = pltpu.async_copy(tids_ref.at[pl.ds(gseg0 * T, nseg_w * T)], ids_v,
                            sem_ids)
    cp_r = pltpu.async_copy(rids_ref.at[pl.ds(gseg0, nseg_w)], rids_v, sem_ids)
    cp_t.wait()
    cp_r.wait()

    def issue(c, ra, rb, rr, sem):
        off = c * (NCSEG * T)
        pltpu.async_copy(tok_ref.at[ids_v.at[pl.ds(off, ROWS_H)]], ra, sem)
        pltpu.async_copy(tok_ref.at[ids_v.at[pl.ds(off + ROWS_H, ROWS_H)]],
                         rb, sem)
        pltpu.async_copy(rul_ref.at[rids_v.at[pl.ds(c * NCSEG, NCSEG)]], rr,
                         sem)

    def drain(ra, rb, rr, sem):
        pltpu.make_async_copy(tok_ref.at[pl.ds(0, ROWS_H)], ra, sem).wait()
        pltpu.make_async_copy(tok_ref.at[pl.ds(0, ROWS_H)], rb, sem).wait()
        pltpu.make_async_copy(rul_ref.at[pl.ds(0, NCSEG)], rr, sem).wait()

    def wait_out(outv, semo):
        pltpu.make_async_copy(outv, out_ref.at[pl.ds(0, NCSEG)], semo).wait()

    def pool_store(c, ra, rb, rr, outv, semo):
        _pool_half(ra, rr, outv, 0)
        _pool_half(rb, rr, outv, HALF)
        pltpu.async_copy(outv, out_ref.at[pl.ds(seg0 + c * NCSEG, NCSEG)],
                         semo)

    issue(0, rows0a, rows0b, rr0, sem0)

    @pl.loop(0, nchunks, step=2)
    def _chunk(c):
        issue(c + 1, rows1a, rows1b, rr1, sem1)
        drain(rows0a, rows0b, rr0, sem0)

        @pl.when(c >= 2)
        def _():
            wait_out(outv0, semo0)

        pool_store(c, rows0a, rows0b, rr0, outv0, semo0)

        @pl.when(c + 2 < nchunks)
        def _():
            issue(c + 2, rows0a, rows0b, rr0, sem0)

        drain(rows1a, rows1b, rr1, sem1)

        @pl.when(c >= 2)
        def _():
            wait_out(outv1, semo1)

        pool_store(c + 1, rows1a, rows1b, rr1, outv1, semo1)

    wait_out(outv0, semo0)
    wait_out(outv1, semo1)


def _gather_pool(tids, rids, token_embeds, rule_embeds, nseg, base_seg):
    info = plsc.get_sparse_core_info()
    nw = info.num_cores * info.num_subcores
    nseg_w = nseg // nw
    mesh = plsc.VectorSubcoreMesh(core_axis_name="c", subcore_axis_name="s")
    kfn = pl.kernel(
        functools.partial(_sc_body, nseg_w, base_seg),
        out_type=jax.ShapeDtypeStruct((nseg, D3), jnp.float32),
        mesh=mesh,
        scratch_types=[
            pltpu.VMEM((nseg_w * T,), jnp.int32),
            pltpu.VMEM((nseg_w,), jnp.int32),
            pltpu.VMEM((ROWS_H, D), jnp.float32),
            pltpu.VMEM((ROWS_H, D), jnp.float32),
            pltpu.VMEM((ROWS_H, D), jnp.float32),
            pltpu.VMEM((ROWS_H, D), jnp.float32),
            pltpu.VMEM((NCSEG, D), jnp.float32),
            pltpu.VMEM((NCSEG, D), jnp.float32),
            pltpu.VMEM((NCSEG, D3), jnp.float32),
            pltpu.VMEM((NCSEG, D3), jnp.float32),
            pltpu.SemaphoreType.DMA,
            pltpu.SemaphoreType.DMA,
            pltpu.SemaphoreType.DMA,
            pltpu.SemaphoreType.DMA,
            pltpu.SemaphoreType.DMA,
        ],
        compiler_params=pltpu.CompilerParams(use_tc_tiling_on_sc=True),
    )
    return kfn(tids, rids, token_embeds, rule_embeds)


MM_BB = 16
NSLICE = 2


def _mm_compute(x_ref, wt_ref, b_ref, o_ref):
    wt = wt_ref[...]
    bias = b_ref[...]
    for j in range(MM_BB):
        x = jnp.maximum(x_ref[pl.ds(j * L, L), :], 0.0)
        o_ref[j] = (
            jnp.dot(x, wt, preferred_element_type=jnp.float32) + bias
        )


def _mm_body(x_ref, wt_ref, b_ref, o_ref):
    _mm_compute(x_ref, wt_ref, b_ref, o_ref)


def _mm_body_alias(x_ref, wt_ref, b_ref, prev_ref, o_ref):
    del prev_ref
    _mm_compute(x_ref, wt_ref, b_ref, o_ref)


def _project_slice(x, wt, b2, prev, blk0, bs):
    nblk = bs // MM_BB
    in_specs = [
        pl.BlockSpec((MM_BB * L, D3), lambda i: (i, 0)),
        pl.BlockSpec((D3, D), lambda i: (0, 0)),
        pl.BlockSpec((1, D), lambda i: (0, 0)),
    ]
    args = (x, wt, b2)
    kwargs = {}
    body = _mm_body
    if prev is not None:
        in_specs.append(pl.BlockSpec(memory_space=pl.ANY))
        args = args + (prev,)
        kwargs["input_output_aliases"] = {3: 0}
        body = _mm_body_alias
    return pl.pallas_call(
        body,
        grid=(nblk,),
        in_specs=in_specs,
        out_specs=pl.BlockSpec(
            (MM_BB, L, D), lambda i, blk0=blk0: (i + blk0, 0, 0)
        ),
        out_shape=jax.ShapeDtypeStruct((B, L, D), jnp.float32),
        **kwargs,
    )(*args)


SLICES = (256, 512, 256)


def kernel(rule_token_ids, rule_ids, token_embeds, rule_embeds, W, b):
    wt = W.T
    b2 = b.reshape(1, D)
    out = None
    off = 0
    for bs in SLICES:
        tids = rule_token_ids[off:off + bs].reshape(-1)
        rids = rule_ids[off:off + bs].reshape(-1)
        concat = _gather_pool(tids.astype(jnp.int32), rids.astype(jnp.int32),
                              token_embeds, rule_embeds, bs * L, 0)
        out = _project_slice(concat, wt, b2, out, off // MM_BB, bs)
        off += bs
    return out


# speedup vs baseline: 1.0975x; 1.0091x over previous
import functools

import jax
import jax.numpy as jnp
from jax import lax
from jax.experimental import pallas as pl
from jax.experimental.pallas import tpu as pltpu
from jax.experimental.pallas import tpu_sc as plsc

B, L, T, D = 1024, 50, 20, 128
N = B * L
D3 = 3 * D
NCSEG = 8
HALF = NCSEG // 2
ROWS_H = HALF * T


def _pool_half(rows_v, rrows_v, out_v, s_off):
    @pl.loop(0, HALF)
    def _seg(s):
        base = s * T
        so = s + s_off
        accs = []
        for j in range(D // 16):
            sl = pl.ds(j * 16, 16)
            parent = rows_v[base, sl]
            out_v[so, pl.ds(j * 16, 16)] = parent
            accs.append(parent)
        for t in range(1, T):
            for j in range(D // 16):
                accs[j] = accs[j] + rows_v[base + t, pl.ds(j * 16, 16)]
        for j in range(D // 16):
            out_v[so, pl.ds(D + j * 16, 16)] = accs[j]
            out_v[so, pl.ds(2 * D + j * 16, 16)] = rrows_v[so, pl.ds(j * 16, 16)]


def _sc_body(nseg_w, base_seg, tids_ref, rids_ref, tok_ref, rul_ref, out_ref,
             ids_v, rids_v, rows0a, rows0b, rows1a, rows1b, rr0, rr1,
             outv0, outv1, sem_ids, sem0, sem1, semo0, semo1):
    nc = 2
    wid = lax.axis_index("s") * nc + lax.axis_index("c")
    seg0 = wid * nseg_w
    gseg0 = base_seg + seg0
    nchunks = nseg_w // NCSEG

    cp_t = pltpu.async_copy(tids_ref.at[pl.ds(gseg0 * T, nseg_w * T)], ids_v,
                            sem_ids)
    cp_r = pltpu.async_copy(rids_ref.at[pl.ds(gseg0, nseg_w)], rids_v, sem_ids)
    cp_t.wait()
    cp_r.wait()

    def issue(c, ra, rb, rr, sem):
        off = c * (NCSEG * T)
        pltpu.async_copy(tok_ref.at[ids_v.at[pl.ds(off, ROWS_H)]], ra, sem)
        pltpu.async_copy(tok_ref.at[ids_v.at[pl.ds(off + ROWS_H, ROWS_H)]],
                         rb, sem)
        pltpu.async_copy(rul_ref.at[rids_v.at[pl.ds(c * NCSEG, NCSEG)]], rr,
                         sem)

    def drain(ra, rb, rr, sem):
        pltpu.make_async_copy(tok_ref.at[pl.ds(0, ROWS_H)], ra, sem).wait()
        pltpu.make_async_copy(tok_ref.at[pl.ds(0, ROWS_H)], rb, sem).wait()
        pltpu.make_async_copy(rul_ref.at[pl.ds(0, NCSEG)], rr, sem).wait()

    def wait_out(outv, semo):
        pltpu.make_async_copy(outv, out_ref.at[pl.ds(0, NCSEG)], semo).wait()

    def pool_store(c, ra, rb, rr, outv, semo):
        _pool_half(ra, rr, outv, 0)
        _pool_half(rb, rr, outv, HALF)
        pltpu.async_copy(outv, out_ref.at[pl.ds(seg0 + c * NCSEG, NCSEG)],
                         semo)

    issue(0, rows0a, rows0b, rr0, sem0)
    nch2 = nchunks - (nchunks % 2)

    @pl.loop(0, nch2, step=2)
    def _chunk(c):
        issue(c + 1, rows1a, rows1b, rr1, sem1)
        drain(rows0a, rows0b, rr0, sem0)

        @pl.when(c >= 2)
        def _():
            wait_out(outv0, semo0)

        pool_store(c, rows0a, rows0b, rr0, outv0, semo0)

        @pl.when(c + 2 < nchunks)
        def _():
            issue(c + 2, rows0a, rows0b, rr0, sem0)

        drain(rows1a, rows1b, rr1, sem1)

        @pl.when(c >= 2)
        def _():
            wait_out(outv1, semo1)

        pool_store(c + 1, rows1a, rows1b, rr1, outv1, semo1)

    if nchunks % 2:
        drain(rows0a, rows0b, rr0, sem0)
        wait_out(outv0, semo0)
        pool_store(nchunks - 1, rows0a, rows0b, rr0, outv0, semo0)

    wait_out(outv0, semo0)
    wait_out(outv1, semo1)


def _gather_pool(tids, rids, token_embeds, rule_embeds, nseg, base_seg):
    info = plsc.get_sparse_core_info()
    nw = info.num_cores * info.num_subcores
    nseg_w = nseg // nw
    mesh = plsc.VectorSubcoreMesh(core_axis_name="c", subcore_axis_name="s")
    kfn = pl.kernel(
        functools.partial(_sc_body, nseg_w, base_seg),
        out_type=jax.ShapeDtypeStruct((nseg, D3), jnp.float32),
        mesh=mesh,
        scratch_types=[
            pltpu.VMEM((nseg_w * T,), jnp.int32),
            pltpu.VMEM((nseg_w,), jnp.int32),
            pltpu.VMEM((ROWS_H, D), jnp.float32),
            pltpu.VMEM((ROWS_H, D), jnp.float32),
            pltpu.VMEM((ROWS_H, D), jnp.float32),
            pltpu.VMEM((ROWS_H, D), jnp.float32),
            pltpu.VMEM((NCSEG, D), jnp.float32),
            pltpu.VMEM((NCSEG, D), jnp.float32),
            pltpu.VMEM((NCSEG, D3), jnp.float32),
            pltpu.VMEM((NCSEG, D3), jnp.float32),
            pltpu.SemaphoreType.DMA,
            pltpu.SemaphoreType.DMA,
            pltpu.SemaphoreType.DMA,
            pltpu.SemaphoreType.DMA,
            pltpu.SemaphoreType.DMA,
        ],
        compiler_params=pltpu.CompilerParams(use_tc_tiling_on_sc=True),
    )
    return kfn(tids, rids, token_embeds, rule_embeds)


MM_BB = 16
NSLICE = 2


def _mm_compute(x_ref, wt_ref, b_ref, o_ref):
    wt = wt_ref[...]
    bias = b_ref[...]
    for j in range(MM_BB):
        x = jnp.maximum(x_ref[pl.ds(j * L, L), :], 0.0)
        o_ref[j] = (
            jnp.dot(x, wt, preferred_element_type=jnp.float32) + bias
        )


def _mm_body(x_ref, wt_ref, b_ref, o_ref):
    _mm_compute(x_ref, wt_ref, b_ref, o_ref)


def _mm_body_alias(x_ref, wt_ref, b_ref, prev_ref, o_ref):
    del prev_ref
    _mm_compute(x_ref, wt_ref, b_ref, o_ref)


def _project_slice(x, wt, b2, prev, blk0, bs):
    nblk = bs // MM_BB
    in_specs = [
        pl.BlockSpec((MM_BB * L, D3), lambda i: (i, 0)),
        pl.BlockSpec((D3, D), lambda i: (0, 0)),
        pl.BlockSpec((1, D), lambda i: (0, 0)),
    ]
    args = (x, wt, b2)
    kwargs = {}
    body = _mm_body
    if prev is not None:
        in_specs.append(pl.BlockSpec(memory_space=pl.ANY))
        args = args + (prev,)
        kwargs["input_output_aliases"] = {3: 0}
        body = _mm_body_alias
    return pl.pallas_call(
        body,
        grid=(nblk,),
        in_specs=in_specs,
        out_specs=pl.BlockSpec(
            (MM_BB, L, D), lambda i, blk0=blk0: (i + blk0, 0, 0)
        ),
        out_shape=jax.ShapeDtypeStruct((B, L, D), jnp.float32),
        **kwargs,
    )(*args)


SLICES = (128, 512, 256, 128)


def kernel(rule_token_ids, rule_ids, token_embeds, rule_embeds, W, b):
    wt = W.T
    b2 = b.reshape(1, D)
    out = None
    off = 0
    for bs in SLICES:
        tids = rule_token_ids[off:off + bs].reshape(-1)
        rids = rule_ids[off:off + bs].reshape(-1)
        concat = _gather_pool(tids.astype(jnp.int32), rids.astype(jnp.int32),
                              token_embeds, rule_embeds, bs * L, 0)
        out = _project_slice(concat, wt, b2, out, off // MM_BB, bs)
        off += bs
    return out
